# baseline (speedup 1.0000x reference)
import functools

import jax
import jax.numpy as jnp
from jax import lax
from jax.experimental import pallas as pl
from jax.experimental.pallas import tpu as pltpu
from jax.experimental.pallas import tpu_sc as plsc

BATCH = 16384
DIM = 64
L = 16
NC, NS = 2, 16
NW = NC * NS
CHUNK = BATCH // NW
NIDX = 4
IDXW = CHUNK // NIDX
NGROUPS = CHUNK // L

_mesh = plsc.VectorSubcoreMesh(core_axis_name="c", subcore_axis_name="s")


@functools.partial(
    pl.kernel,
    out_type=jax.ShapeDtypeStruct((BATCH,), jnp.float32),
    mesh=_mesh,
    compiler_params=pltpu.CompilerParams(
        needs_layout_passes=False, use_tc_tiling_on_sc=False),
    scratch_types=[
        pltpu.VMEM((NIDX, IDXW), jnp.int32),
        pltpu.VMEM((CHUNK, DIM), jnp.float32),
        pltpu.VMEM((CHUNK, DIM), jnp.float32),
        pltpu.VMEM((CHUNK, DIM), jnp.float32),
        pltpu.VMEM((CHUNK,), jnp.float32),
        pltpu.SemaphoreType.DMA,
    ],
)
def _distmult_sc(d1_hbm, d2_hbm, ctx_hbm, rel_hbm, out_hbm,
                 idx_v, rel_v, d1_v, d2_v, out_v, sem):
    wid = lax.axis_index("s") * NC + lax.axis_index("c")
    base = wid * CHUNK

    pltpu.sync_copy(ctx_hbm.at[wid], idx_v)
    copies = []
    for j in range(NIDX):
        copies.append(pltpu.async_copy(
            rel_hbm.at[idx_v.at[j]],
            rel_v.at[pl.ds(j * IDXW, IDXW)], sem))
    copies.append(pltpu.async_copy(d1_hbm.at[pl.ds(base, CHUNK)], d1_v, sem))
    copies.append(pltpu.async_copy(d2_hbm.at[pl.ds(base, CHUNK)], d2_v, sem))
    for c in copies:
        c.wait()

    def group(g, carry):
        rows = lax.iota(jnp.int32, L) + g * L
        acc = jnp.zeros((L,), jnp.float32)
        for d in range(DIM):
            cols = jnp.full((L,), d, jnp.int32)
            a = plsc.load_gather(d1_v, [rows, cols])
            r = plsc.load_gather(rel_v, [rows, cols])
            b = plsc.load_gather(d2_v, [rows, cols])
            acc = acc + a * r * b
        out_v[pl.ds(g * L, L)] = acc
        return carry

    lax.fori_loop(0, NGROUPS, group, 0)
    pltpu.sync_copy(out_v, out_hbm.at[pl.ds(base, CHUNK)])


def kernel(d1_embd, d2_embd, context_ids, drug_1_ids, drug_2_ids, relation):
    ctx = context_ids.astype(jnp.int32).reshape(NW, NIDX, IDXW)
    return _distmult_sc(d1_embd, d2_embd, ctx, relation)


# speedup vs baseline: 1.7683x; 1.7683x over previous
import functools

import jax
import jax.numpy as jnp
from jax import lax
from jax.experimental import pallas as pl
from jax.experimental.pallas import tpu as pltpu
from jax.experimental.pallas import tpu_sc as plsc

BATCH = 16384
DIM = 64
L = 16
NC, NS = 2, 16
NW = NC * NS
CHUNK = BATCH // NW
NSUB = 2
SUB = CHUNK // NSUB
NG = SUB // L

_mesh = plsc.VectorSubcoreMesh(core_axis_name="c", subcore_axis_name="s")


@functools.partial(
    pl.kernel,
    out_type=jax.ShapeDtypeStruct((BATCH,), jnp.float32),
    mesh=_mesh,
    compiler_params=pltpu.CompilerParams(
        needs_layout_passes=False, use_tc_tiling_on_sc=True),
    scratch_types=[
        pltpu.VMEM((CHUNK,), jnp.int32),
        pltpu.VMEM((SUB, DIM), jnp.float32),
        pltpu.VMEM((SUB, DIM), jnp.float32),
        pltpu.VMEM((SUB, DIM), jnp.float32),
        pltpu.VMEM((CHUNK,), jnp.float32),
        pltpu.SemaphoreType.DMA,
        pltpu.SemaphoreType.DMA,
    ],
)
def _distmult_sc(d1_hbm, d2_hbm, ctx_hbm, rel_hbm, out_hbm,
                 idx_v, rel_v, d1_v, d2_v, out_v, gsem, dsem):
    wid = lax.axis_index("s") * NC + lax.axis_index("c")
    base = wid * CHUNK

    pltpu.sync_copy(ctx_hbm.at[pl.ds(base, CHUNK)], idx_v)

    for h in range(NSUB):
        hoff = h * SUB
        cp1 = pltpu.async_copy(
            d1_hbm.at[pl.ds(base + hoff, SUB)], d1_v, dsem)
        cp2 = pltpu.async_copy(
            d2_hbm.at[pl.ds(base + hoff, SUB)], d2_v, dsem)

        def issue(i, carry, hoff=hoff):
            iv = idx_v[pl.ds(hoff + i * L, L)]
            for j in range(L):
                pltpu.async_copy(
                    rel_hbm.at[iv[j]], rel_v.at[i * L + j], gsem)
            return carry

        lax.fori_loop(0, SUB // L, issue, 0)
        pltpu.make_async_copy(rel_hbm.at[pl.ds(0, SUB)], rel_v, gsem).wait()
        cp1.wait()
        cp2.wait()

        def group(g, carry, hoff=hoff):
            outv = jnp.zeros((L,), jnp.float32)
            for j in range(L):
                r = g * L + j
                acc = jnp.zeros((L,), jnp.float32)
                for c in range(DIM // L):
                    s = pl.ds(c * L, L)
                    acc += d1_v[r, s] * rel_v[r, s] * d2_v[r, s]
                lane = lax.iota(jnp.int32, L)
                outv = jnp.where(lane == j, jnp.sum(acc), outv)
            out_v[pl.ds(hoff + g * L, L)] = outv
            return carry

        lax.fori_loop(0, NG, group, 0)

    pltpu.sync_copy(out_v, out_hbm.at[pl.ds(base, CHUNK)])


def kernel(d1_embd, d2_embd, context_ids, drug_1_ids, drug_2_ids, relation):
    return _distmult_sc(
        d1_embd, d2_embd, context_ids.astype(jnp.int32), relation)
